# baseline (speedup 1.0000x reference)
import functools

import jax
import jax.numpy as jnp
from jax import lax
from jax.experimental import pallas as pl
from jax.experimental.pallas import tpu as pltpu
from jax.experimental.pallas import tpu_sc as plsc

N = 10000
D = 128
E = 160000
NEG_INF = float("-inf")

NUM_TECS = 32
TECS_PER_ET = 16
NPAD = 10240
ROWS_PER_TEC = NPAD // TECS_PER_ET
CHUNK = 2000
VPC = CHUNK // 16
GB = 128
NCHUNK = E // CHUNK

ROW_BLK = 1000


def _proj_body(xu, xi, wsu, wtu, wsi, wti, sxui, txui, sxiu, txiu):
    a = xu[...]
    b = xi[...]
    f32 = jnp.float32
    sxui[...] = jnp.dot(a, wsu[...], preferred_element_type=f32)
    txui[...] = jnp.dot(b, wtu[...], preferred_element_type=f32)
    sxiu[...] = jnp.dot(b, wsi[...], preferred_element_type=f32)
    txiu[...] = jnp.dot(a, wti[...], preferred_element_type=f32)


def _proj(xu, xi, wsu, wtu, wsi, wti):
    grid = (N // ROW_BLK,)
    xspec = pl.BlockSpec((ROW_BLK, D), lambda i: (i, 0))
    wspec = pl.BlockSpec((D, D), lambda i: (0, 0))
    oshape = jax.ShapeDtypeStruct((N, D), jnp.float32)
    return pl.pallas_call(
        _proj_body,
        grid=grid,
        in_specs=[xspec, xspec, wspec, wspec, wspec, wspec],
        out_specs=[xspec, xspec, xspec, xspec],
        out_shape=[oshape, oshape, oshape, oshape],
    )(xu, xi, wsu, wtu, wsi, wti)


def _lstm_one(tx, agg, wih, whh, b):
    g = (jnp.dot(tx, wih, preferred_element_type=jnp.float32)
         + jnp.dot(agg, whh, preferred_element_type=jnp.float32) + b)
    i = jax.nn.sigmoid(g[:, 0 * D:1 * D])
    f = jax.nn.sigmoid(g[:, 1 * D:2 * D])
    gg = jnp.tanh(g[:, 2 * D:3 * D])
    o = jax.nn.sigmoid(g[:, 3 * D:4 * D])
    c2 = f * agg + i * gg
    return jnp.maximum(o * jnp.tanh(c2), 0.0)


def _lstm_body(txui, aggui, wihu, whhu, bu, txiu, aggiu, wihi, whhi, bi,
               xin, xun):
    xin[...] = _lstm_one(txui[...], aggui[...], wihu[...], whhu[...], bu[...])
    xun[...] = _lstm_one(txiu[...], aggiu[...], wihi[...], whhi[...], bi[...])


def _lstm(txui, aggui, wihu, whhu, bu, txiu, aggiu, wihi, whhi, bi):
    grid = (N // ROW_BLK,)
    xspec = pl.BlockSpec((ROW_BLK, D), lambda i: (i, 0))
    wspec = pl.BlockSpec((D, 4 * D), lambda i: (0, 0))
    bspec = pl.BlockSpec((1, 4 * D), lambda i: (0, 0))
    oshape = jax.ShapeDtypeStruct((N, D), jnp.float32)
    return pl.pallas_call(
        _lstm_body,
        grid=grid,
        in_specs=[xspec, xspec, wspec, wspec, bspec,
                  xspec, xspec, wspec, wspec, bspec],
        out_specs=[xspec, xspec],
        out_shape=[oshape, oshape],
    )(txui, aggui, wihu, whhu, bu, txiu, aggiu, wihi, whhi, bi)


def _head_body(xu1, xu2, xi1, xi2, wu, bu, wi, bi, xum, xim, ou, oi):
    a1 = xu1[...]
    a2 = xu2[...]
    b1 = xi1[...]
    b2 = xi2[...]
    xum[...] = (a1 + a2) * 0.5
    xim[...] = (b1 + b2) * 0.5
    ou[...] = jnp.sum(a2 * wu[...], axis=1, keepdims=True) + bu[...]
    oi[...] = jnp.sum(b2 * wi[...], axis=1, keepdims=True) + bi[...]


def _head(xu1, xu2, xi1, xi2, wu, bu, wi, bi):
    grid = (N // ROW_BLK,)
    xspec = pl.BlockSpec((ROW_BLK, D), lambda i: (i, 0))
    wspec = pl.BlockSpec((1, D), lambda i: (0, 0))
    sspec = pl.BlockSpec((1, 1), lambda i: (0, 0))
    ospec = pl.BlockSpec((ROW_BLK, 1), lambda i: (i, 0))
    return pl.pallas_call(
        _head_body,
        grid=grid,
        in_specs=[xspec, xspec, xspec, xspec, wspec, sspec, wspec, sspec],
        out_specs=[xspec, xspec, ospec, ospec],
        out_shape=[jax.ShapeDtypeStruct((N, D), jnp.float32),
                   jax.ShapeDtypeStruct((N, D), jnp.float32),
                   jax.ShapeDtypeStruct((N, 1), jnp.float32),
                   jax.ShapeDtypeStruct((N, 1), jnp.float32)],
    )(xu1, xu2, xi1, xi2, wu, bu, wi, bi)


def _segmax_body(sxcat, ecat, outcat, accv, ebufa, ebufb,
                 srclv, dstlv, rowsa, rowsb, esema, esemb, dsema, dsemb):
    cid = lax.axis_index("c")
    sid = lax.axis_index("s")
    wid = sid * 2 + cid
    et = wid // TECS_PER_ET
    t = wid % TECS_PER_ET
    lo = t * ROWS_PER_TEC
    hi = lo + ROWS_PER_TEC
    ebase = et * 2 * E
    obase = et * NPAD + lo

    iota = lax.iota(jnp.int32, 16)
    neg = jnp.full((16,), NEG_INF, dtype=jnp.float32)
    dummy = jnp.full((16,), ROWS_PER_TEC, dtype=jnp.int32)

    def _init(r, _):
        for g in range(8):
            accv[r, pl.ds(g * 16, 16)] = neg
        return 0
    lax.fori_loop(0, ROWS_PER_TEC, _init, 0, unroll=8)

    def _fire_edges(ci, ebuf, esem):
        pltpu.async_copy(ecat.at[pl.ds(ebase + ci * 2 * CHUNK, 2 * CHUNK)],
                         ebuf, esem)

    def _wait_edges(ebuf, esem):
        pltpu.make_async_copy(ecat.at[pl.ds(0, 2 * CHUNK)], ebuf, esem).wait()

    def _scan_chunk(ebuf):
        def _scan(i, m):
            s = ebuf[pl.ds(i * 16, 16)]
            d = ebuf[pl.ds(CHUNK + i * 16, 16)]
            msk = (d >= lo) & (d < hi)
            cnt = plsc.all_reduce_population_count(msk)[0]
            pos = m + plsc.cumsum(msk.astype(jnp.int32)) - 1
            plsc.store_scatter(srclv, [pos], s, mask=msk)
            plsc.store_scatter(dstlv, [pos], d - lo, mask=msk)
            return m + cnt
        return lax.fori_loop(0, VPC, _scan, jnp.int32(0), unroll=4)

    def _fire_rows(k, rows, dsem):
        pltpu.async_copy(sxcat.at[srclv.at[pl.ds(k * GB, GB)]], rows, dsem)

    def _wait_rows(rows, dsem):
        pltpu.make_async_copy(sxcat.at[pl.ds(0, GB)], rows, dsem).wait()

    def _process(k, rows):
        def _edge(e, _):
            dl = dstlv[pl.ds(k * GB + e, 16)][0]
            for g in range(8):
                sl = pl.ds(g * 16, 16)
                accv[dl, sl] = jnp.maximum(accv[dl, sl], rows[e, sl])
            return 0
        lax.fori_loop(0, GB, _edge, 0, unroll=8)

    def _do_chunk(ci, ebuf, esem):
        m = _scan_chunk(ebuf)
        @pl.when(ci + 2 < NCHUNK)
        def _():
            _fire_edges(ci + 2, ebuf, esem)
        for j in range(GB // 16):
            plsc.store_scatter(srclv, [m + j * 16 + iota],
                               wid * GB + j * 16 + iota)
            plsc.store_scatter(dstlv, [m + j * 16 + iota], dummy)
        npass = (m + (GB - 1)) // GB

        @pl.when(npass > 0)
        def _():
            _fire_rows(0, rowsa, dsema)

            def _pass(k, _):
                even = (k % 2) == 0
                more = k + 1 < npass

                @pl.when(more & even)
                def _():
                    _fire_rows(k + 1, rowsb, dsemb)

                @pl.when(more & jnp.logical_not(even))
                def _():
                    _fire_rows(k + 1, rowsa, dsema)

                @pl.when(even)
                def _():
                    _wait_rows(rowsa, dsema)
                    _process(k, rowsa)

                @pl.when(jnp.logical_not(even))
                def _():
                    _wait_rows(rowsb, dsemb)
                    _process(k, rowsb)
                return 0
            lax.fori_loop(0, npass, _pass, 0)

    _fire_edges(0, ebufa, esema)
    _fire_edges(1, ebufb, esemb)

    def _two_chunks(kk, _):
        ci = kk * 2
        _wait_edges(ebufa, esema)
        _do_chunk(ci, ebufa, esema)
        _wait_edges(ebufb, esemb)
        _do_chunk(ci + 1, ebufb, esemb)
        return 0
    lax.fori_loop(0, NCHUNK // 2, _two_chunks, 0)

    def _fix(r, _):
        for g in range(8):
            sl = pl.ds(g * 16, 16)
            v = accv[r, sl]
            accv[r, sl] = jnp.where(v == NEG_INF, 0.0, v)
        return 0
    lax.fori_loop(0, ROWS_PER_TEC, _fix, 0, unroll=8)
    pltpu.sync_copy(accv.at[pl.ds(0, ROWS_PER_TEC)],
                    outcat.at[pl.ds(obase, ROWS_PER_TEC)])


def _interleave(src, dst):
    s = src.reshape(NCHUNK, 1, CHUNK)
    d = dst.reshape(NCHUNK, 1, CHUNK)
    return jnp.concatenate([s, d], axis=1).reshape(-1)


def _segmax_pair(sx_ui, sx_iu, src_ui, dst_ui, src_iu, dst_iu):
    sxcat = jnp.concatenate([sx_ui, sx_iu], axis=0)
    ecat = jnp.concatenate([_interleave(src_ui, dst_ui),
                            _interleave(src_iu + N, dst_iu)])
    mesh = plsc.VectorSubcoreMesh(core_axis_name="c", subcore_axis_name="s")
    f = pl.kernel(
        _segmax_body,
        mesh=mesh,
        compiler_params=pltpu.CompilerParams(needs_layout_passes=False),
        out_type=jax.ShapeDtypeStruct((2 * NPAD, D), jnp.float32),
        scratch_types=[
            pltpu.VMEM((ROWS_PER_TEC + 8, D), jnp.float32),
            pltpu.VMEM((2 * CHUNK,), jnp.int32),
            pltpu.VMEM((2 * CHUNK,), jnp.int32),
            pltpu.VMEM((CHUNK + GB,), jnp.int32),
            pltpu.VMEM((CHUNK + GB,), jnp.int32),
            pltpu.VMEM((GB, D), jnp.float32),
            pltpu.VMEM((GB, D), jnp.float32),
            pltpu.SemaphoreType.DMA,
            pltpu.SemaphoreType.DMA,
            pltpu.SemaphoreType.DMA,
            pltpu.SemaphoreType.DMA,
        ],
    )
    aggcat = f(sxcat, ecat)
    return aggcat[:N], aggcat[NPAD:NPAD + N]


def kernel(x_user, x_item, ei_ui, ei_iu, params):
    p = params
    src_ui, dst_ui = ei_ui[0], ei_ui[1]
    src_iu, dst_iu = ei_iu[0], ei_iu[1]

    xu, xi = x_user, x_item
    layer_u = []
    layer_i = []
    for l in range(2):
        sx_ui, tx_ui, sx_iu, tx_iu = _proj(
            xu, xi,
            p["Wsrc_%d_ui" % l], p["Wtgt_%d_ui" % l],
            p["Wsrc_%d_iu" % l], p["Wtgt_%d_iu" % l])
        agg_ui, agg_iu = _segmax_pair(sx_ui, sx_iu,
                                      src_ui, dst_ui, src_iu, dst_iu)
        bu = (p["bih_%d_ui" % l] + p["bhh_%d_ui" % l]).reshape(1, 4 * D)
        bi = (p["bih_%d_iu" % l] + p["bhh_%d_iu" % l]).reshape(1, 4 * D)
        xi_n, xu_n = _lstm(tx_ui, agg_ui,
                           p["Wih_%d_ui" % l].T, p["Whh_%d_ui" % l].T, bu,
                           tx_iu, agg_iu,
                           p["Wih_%d_iu" % l].T, p["Whh_%d_iu" % l].T, bi)
        xu, xi = xu_n, xi_n
        layer_u.append(xu)
        layer_i.append(xi)

    xum, xim, ou, oi = _head(
        layer_u[0], layer_u[1], layer_i[0], layer_i[1],
        p["Wout_user"].reshape(1, D), p["bout_user"].reshape(1, 1),
        p["Wout_item"].reshape(1, D), p["bout_item"].reshape(1, 1))
    return (xum, xim, ou, oi)


# speedup vs baseline: 2.7331x; 2.7331x over previous
import functools

import jax
import jax.numpy as jnp
from jax import lax
from jax.experimental import pallas as pl
from jax.experimental.pallas import tpu as pltpu
from jax.experimental.pallas import tpu_sc as plsc

N = 10000
D = 128
E = 160000
NEG_INF = float("-inf")

NUM_TECS = 32
TECS_PER_ET = 16
NPAD = 10240
ROWS_PER_TEC = NPAD // TECS_PER_ET
CHUNK = 1600
VPC = CHUNK // 16
GB = 64
NCHUNK = E // CHUNK

ROW_BLK = 1000


def _proj_body(xu, xi, wsu, wtu, wsi, wti, sxui, txui, sxiu, txiu):
    a = xu[...]
    b = xi[...]
    f32 = jnp.float32
    sxui[...] = jnp.dot(a, wsu[...], preferred_element_type=f32)
    txui[...] = jnp.dot(b, wtu[...], preferred_element_type=f32)
    sxiu[...] = jnp.dot(b, wsi[...], preferred_element_type=f32)
    txiu[...] = jnp.dot(a, wti[...], preferred_element_type=f32)


def _proj(xu, xi, wsu, wtu, wsi, wti):
    grid = (N // ROW_BLK,)
    xspec = pl.BlockSpec((ROW_BLK, D), lambda i: (i, 0))
    wspec = pl.BlockSpec((D, D), lambda i: (0, 0))
    oshape = jax.ShapeDtypeStruct((N, D), jnp.float32)
    return pl.pallas_call(
        _proj_body,
        grid=grid,
        in_specs=[xspec, xspec, wspec, wspec, wspec, wspec],
        out_specs=[xspec, xspec, xspec, xspec],
        out_shape=[oshape, oshape, oshape, oshape],
    )(xu, xi, wsu, wtu, wsi, wti)


def _lstm_one(tx, agg, wih, whh, b):
    g = (jnp.dot(tx, wih, preferred_element_type=jnp.float32)
         + jnp.dot(agg, whh, preferred_element_type=jnp.float32) + b)
    i = jax.nn.sigmoid(g[:, 0 * D:1 * D])
    f = jax.nn.sigmoid(g[:, 1 * D:2 * D])
    gg = jnp.tanh(g[:, 2 * D:3 * D])
    o = jax.nn.sigmoid(g[:, 3 * D:4 * D])
    c2 = f * agg + i * gg
    return jnp.maximum(o * jnp.tanh(c2), 0.0)


def _lstm_body(txui, aggui, wihu, whhu, bu, txiu, aggiu, wihi, whhi, bi,
               xin, xun):
    xin[...] = _lstm_one(txui[...], aggui[...], wihu[...], whhu[...], bu[...])
    xun[...] = _lstm_one(txiu[...], aggiu[...], wihi[...], whhi[...], bi[...])


def _lstm(txui, aggui, wihu, whhu, bu, txiu, aggiu, wihi, whhi, bi):
    grid = (N // ROW_BLK,)
    xspec = pl.BlockSpec((ROW_BLK, D), lambda i: (i, 0))
    wspec = pl.BlockSpec((D, 4 * D), lambda i: (0, 0))
    bspec = pl.BlockSpec((1, 4 * D), lambda i: (0, 0))
    oshape = jax.ShapeDtypeStruct((N, D), jnp.float32)
    return pl.pallas_call(
        _lstm_body,
        grid=grid,
        in_specs=[xspec, xspec, wspec, wspec, bspec,
                  xspec, xspec, wspec, wspec, bspec],
        out_specs=[xspec, xspec],
        out_shape=[oshape, oshape],
    )(txui, aggui, wihu, whhu, bu, txiu, aggiu, wihi, whhi, bi)


def _head_body(xu1, xu2, xi1, xi2, wu, bu, wi, bi, xum, xim, ou, oi):
    a1 = xu1[...]
    a2 = xu2[...]
    b1 = xi1[...]
    b2 = xi2[...]
    xum[...] = (a1 + a2) * 0.5
    xim[...] = (b1 + b2) * 0.5
    ou[...] = jnp.sum(a2 * wu[...], axis=1, keepdims=True) + bu[...]
    oi[...] = jnp.sum(b2 * wi[...], axis=1, keepdims=True) + bi[...]


def _head(xu1, xu2, xi1, xi2, wu, bu, wi, bi):
    grid = (N // ROW_BLK,)
    xspec = pl.BlockSpec((ROW_BLK, D), lambda i: (i, 0))
    wspec = pl.BlockSpec((1, D), lambda i: (0, 0))
    sspec = pl.BlockSpec((1, 1), lambda i: (0, 0))
    ospec = pl.BlockSpec((ROW_BLK, 1), lambda i: (i, 0))
    return pl.pallas_call(
        _head_body,
        grid=grid,
        in_specs=[xspec, xspec, xspec, xspec, wspec, sspec, wspec, sspec],
        out_specs=[xspec, xspec, ospec, ospec],
        out_shape=[jax.ShapeDtypeStruct((N, D), jnp.float32),
                   jax.ShapeDtypeStruct((N, D), jnp.float32),
                   jax.ShapeDtypeStruct((N, 1), jnp.float32),
                   jax.ShapeDtypeStruct((N, 1), jnp.float32)],
    )(xu1, xu2, xi1, xi2, wu, bu, wi, bi)


def _segmax_body(sxcat, ecat, outcat, accv, ebufa, ebufb,
                 srcla, dstla, srclb, dstlb, ra0, ra1, rb0, rb1,
                 esema, esemb, sa0, sa1, sb0, sb1):
    cid = lax.axis_index("c")
    sid = lax.axis_index("s")
    et = cid
    t = sid
    lo = t * ROWS_PER_TEC
    hi = lo + ROWS_PER_TEC
    ebase = et * 2 * E
    obase = et * NPAD + lo

    iota = lax.iota(jnp.int32, 16)
    neg = jnp.full((16,), NEG_INF, dtype=jnp.float32)
    dummy = jnp.full((16,), ROWS_PER_TEC, dtype=jnp.int32)

    def _init(r, _):
        for g in range(8):
            accv[r, pl.ds(g * 16, 16)] = neg
        return 0
    lax.fori_loop(0, ROWS_PER_TEC, _init, 0, unroll=8)

    def _fire_edges(ci, ebuf, esem):
        pltpu.async_copy(ecat.at[pl.ds(ebase + ci * 2 * CHUNK, 2 * CHUNK)],
                         ebuf, esem)

    def _wait_edges(ebuf, esem):
        pltpu.make_async_copy(ecat.at[pl.ds(0, 2 * CHUNK)], ebuf, esem).wait()

    def _scan_chunk(ebuf, srcl, dstl):
        def _scan(i, m):
            s = ebuf[pl.ds(i * 16, 16)]
            d = ebuf[pl.ds(CHUNK + i * 16, 16)]
            msk = (d >= lo) & (d < hi)
            cnt = plsc.all_reduce_population_count(msk)[0]
            plsc.store_compressed(srcl.at[pl.ds(m, 16)], s, mask=msk)
            plsc.store_compressed(dstl.at[pl.ds(m, 16)], d - lo, mask=msk)
            return m + cnt
        m = lax.fori_loop(0, VPC, _scan, jnp.int32(0), unroll=4)
        for j in range(GB // 16):
            srcl[pl.ds(m + j * 16, 16)] = t * GB + j * 16 + iota
            dstl[pl.ds(m + j * 16, 16)] = dummy
        return m

    def _fire_rows(srcl, k, rows, dsem):
        pltpu.async_copy(sxcat.at[srcl.at[pl.ds(k * GB, GB)]], rows, dsem)

    def _wait_rows(rows, dsem):
        pltpu.make_async_copy(sxcat.at[pl.ds(0, GB)], rows, dsem).wait()

    def _fire_first2(m, srcl, r0, s0, r1, s1):
        @pl.when(m > 0)
        def _():
            _fire_rows(srcl, 0, r0, s0)

        @pl.when(m > GB)
        def _():
            _fire_rows(srcl, 1, r1, s1)

    def _pbatch(k, dstl, rows):
        def _q(q, _):
            dvec = dstl[pl.ds(k * GB + q * 16, 16)]
            dls = [dvec[e] for e in range(16)]
            for e in range(16):
                dl = dls[e]
                av = [accv[dl, pl.ds(g * 16, 16)] for g in range(8)]
                rv = [rows[q * 16 + e, pl.ds(g * 16, 16)] for g in range(8)]
                for g in range(8):
                    accv[dl, pl.ds(g * 16, 16)] = jnp.maximum(av[g], rv[g])
            return 0
        lax.fori_loop(0, GB // 16, _q, 0)

    def _process(m, srcl, dstl, r0, s0, r1, s1):
        npass = (m + (GB - 1)) // GB

        def _pass(k, _):
            even = (k % 2) == 0

            @pl.when(even)
            def _():
                _wait_rows(r0, s0)
                _pbatch(k, dstl, r0)

                @pl.when(k + 2 < npass)
                def _():
                    _fire_rows(srcl, k + 2, r0, s0)

            @pl.when(jnp.logical_not(even))
            def _():
                _wait_rows(r1, s1)
                _pbatch(k, dstl, r1)

                @pl.when(k + 2 < npass)
                def _():
                    _fire_rows(srcl, k + 2, r1, s1)
            return 0
        lax.fori_loop(0, npass, _pass, 0)

    _fire_edges(0, ebufa, esema)
    _fire_edges(1, ebufb, esemb)

    def _two_chunks(kk, mprev):
        ci = kk * 2
        _wait_edges(ebufa, esema)
        ma = _scan_chunk(ebufa, srcla, dstla)

        @pl.when(ci + 2 < NCHUNK)
        def _():
            _fire_edges(ci + 2, ebufa, esema)
        _fire_first2(ma, srcla, ra0, sa0, ra1, sa1)

        @pl.when(kk > 0)
        def _():
            _process(mprev, srclb, dstlb, rb0, sb0, rb1, sb1)

        _wait_edges(ebufb, esemb)
        mb = _scan_chunk(ebufb, srclb, dstlb)

        @pl.when(ci + 3 < NCHUNK)
        def _():
            _fire_edges(ci + 3, ebufb, esemb)
        _fire_first2(mb, srclb, rb0, sb0, rb1, sb1)

        _process(ma, srcla, dstla, ra0, sa0, ra1, sa1)
        return mb
    mlast = lax.fori_loop(0, NCHUNK // 2, _two_chunks, jnp.int32(0))
    _process(mlast, srclb, dstlb, rb0, sb0, rb1, sb1)

    def _fix(r, _):
        for g in range(8):
            sl = pl.ds(g * 16, 16)
            v = accv[r, sl]
            accv[r, sl] = jnp.where(v == NEG_INF, 0.0, v)
        return 0
    lax.fori_loop(0, ROWS_PER_TEC, _fix, 0, unroll=8)
    pltpu.sync_copy(accv.at[pl.ds(0, ROWS_PER_TEC)],
                    outcat.at[pl.ds(obase, ROWS_PER_TEC)])


def _interleave(src, dst):
    s = src.reshape(NCHUNK, 1, CHUNK)
    d = dst.reshape(NCHUNK, 1, CHUNK)
    return jnp.concatenate([s, d], axis=1).reshape(-1)


def _segmax_pair(sx_ui, sx_iu, src_ui, dst_ui, src_iu, dst_iu):
    sxcat = jnp.concatenate([sx_ui, sx_iu], axis=0)
    ecat = jnp.concatenate([_interleave(src_ui, dst_ui),
                            _interleave(src_iu + N, dst_iu)])
    mesh = plsc.VectorSubcoreMesh(core_axis_name="c", subcore_axis_name="s")
    f = pl.kernel(
        _segmax_body,
        mesh=mesh,
        compiler_params=pltpu.CompilerParams(needs_layout_passes=False),
        out_type=jax.ShapeDtypeStruct((2 * NPAD, D), jnp.float32),
        scratch_types=[
            pltpu.VMEM((ROWS_PER_TEC + 8, D), jnp.float32),
            pltpu.VMEM((2 * CHUNK,), jnp.int32),
            pltpu.VMEM((2 * CHUNK,), jnp.int32),
            pltpu.VMEM((CHUNK + GB,), jnp.int32),
            pltpu.VMEM((CHUNK + GB,), jnp.int32),
            pltpu.VMEM((CHUNK + GB,), jnp.int32),
            pltpu.VMEM((CHUNK + GB,), jnp.int32),
            pltpu.VMEM((GB, D), jnp.float32),
            pltpu.VMEM((GB, D), jnp.float32),
            pltpu.VMEM((GB, D), jnp.float32),
            pltpu.VMEM((GB, D), jnp.float32),
            pltpu.SemaphoreType.DMA,
            pltpu.SemaphoreType.DMA,
            pltpu.SemaphoreType.DMA,
            pltpu.SemaphoreType.DMA,
            pltpu.SemaphoreType.DMA,
            pltpu.SemaphoreType.DMA,
        ],
    )
    aggcat = f(sxcat, ecat)
    return aggcat[:N], aggcat[NPAD:NPAD + N]


def kernel(x_user, x_item, ei_ui, ei_iu, params):
    p = params
    src_ui, dst_ui = ei_ui[0], ei_ui[1]
    src_iu, dst_iu = ei_iu[0], ei_iu[1]

    xu, xi = x_user, x_item
    layer_u = []
    layer_i = []
    for l in range(2):
        sx_ui, tx_ui, sx_iu, tx_iu = _proj(
            xu, xi,
            p["Wsrc_%d_ui" % l], p["Wtgt_%d_ui" % l],
            p["Wsrc_%d_iu" % l], p["Wtgt_%d_iu" % l])
        agg_ui, agg_iu = _segmax_pair(sx_ui, sx_iu,
                                      src_ui, dst_ui, src_iu, dst_iu)
        bu = (p["bih_%d_ui" % l] + p["bhh_%d_ui" % l]).reshape(1, 4 * D)
        bi = (p["bih_%d_iu" % l] + p["bhh_%d_iu" % l]).reshape(1, 4 * D)
        xi_n, xu_n = _lstm(tx_ui, agg_ui,
                           p["Wih_%d_ui" % l].T, p["Whh_%d_ui" % l].T, bu,
                           tx_iu, agg_iu,
                           p["Wih_%d_iu" % l].T, p["Whh_%d_iu" % l].T, bi)
        xu, xi = xu_n, xi_n
        layer_u.append(xu)
        layer_i.append(xi)

    xum, xim, ou, oi = _head(
        layer_u[0], layer_u[1], layer_i[0], layer_i[1],
        p["Wout_user"].reshape(1, D), p["bout_user"].reshape(1, 1),
        p["Wout_item"].reshape(1, D), p["bout_item"].reshape(1, 1))
    return (xum, xim, ou, oi)
